# baseline (speedup 1.0000x reference)
import functools

import jax
import jax.numpy as jnp
from jax import lax
from jax.experimental import pallas as pl
from jax.experimental.pallas import tpu as pltpu
from jax.experimental.pallas import tpu_sc as plsc

N = 10000
E = 320000
D_IN = 128
D_HID = 128
D_OUT = 16

NC, NS = 2, 16
NW = NC * NS
EPW = E // NW
CH = 64
NCHUNK = 160
EPAD = NCHUNK * CH
NPAD = 10240
RPW = NPAD // NS

_MESH = plsc.VectorSubcoreMesh(core_axis_name="c", subcore_axis_name="s")


def _sc_degrees(idx_all):
    @functools.partial(
        pl.kernel,
        out_type=jax.ShapeDtypeStruct((2, 2, NPAD), jnp.float32),
        mesh=_MESH,
        scratch_types=[
            pltpu.VMEM((NCHUNK, CH), jnp.int32),
            pltpu.VMEM((NCHUNK, CH), jnp.int32),
            pltpu.VMEM((CH,), jnp.float32),
            pltpu.VMEM((RPW,), jnp.float32),
            pltpu.VMEM_SHARED((NPAD,), jnp.float32),
            pltpu.VMEM_SHARED((NPAD,), jnp.float32),
        ],
    )
    def k(idx_hbm, out_hbm, src_v, dst_v, ones_v, zer_v, dsrc_sh, ddst_sh):
        c = lax.axis_index("c")
        s = lax.axis_index("s")
        wid = c * NS + s

        @pl.loop(0, CH // 16)
        def _(i):
            ones_v[pl.ds(i * 16, 16)] = jnp.ones((16,), jnp.float32)

        @pl.loop(0, RPW // 16)
        def _(i):
            zer_v[pl.ds(i * 16, 16)] = jnp.zeros((16,), jnp.float32)

        base = s * RPW
        pltpu.sync_copy(zer_v, dsrc_sh.at[pl.ds(base, RPW)])
        pltpu.sync_copy(zer_v, ddst_sh.at[pl.ds(base, RPW)])
        pltpu.sync_copy(idx_hbm.at[0, wid], src_v)
        pltpu.sync_copy(idx_hbm.at[1, wid], dst_v)
        plsc.subcore_barrier()

        @pl.loop(0, NCHUNK)
        def _(j):
            pltpu.sync_copy(ones_v, dsrc_sh.at[src_v.at[j]], add=True)
            pltpu.sync_copy(ones_v, ddst_sh.at[dst_v.at[j]], add=True)

        plsc.subcore_barrier()
        pltpu.sync_copy(dsrc_sh.at[pl.ds(base, RPW)],
                        out_hbm.at[c, 0, pl.ds(base, RPW)])
        pltpu.sync_copy(ddst_sh.at[pl.ds(base, RPW)],
                        out_hbm.at[c, 1, pl.ds(base, RPW)])

    return k(idx_all)


def _sc_aggregate(h, idx_all, width):
    @functools.partial(
        pl.kernel,
        out_type=jax.ShapeDtypeStruct((2, NPAD, width), jnp.float32),
        mesh=_MESH,
        scratch_types=[
            pltpu.VMEM((NCHUNK, CH), jnp.int32),
            pltpu.VMEM((NCHUNK, CH), jnp.int32),
            pltpu.VMEM((CH, width), jnp.float32),
            pltpu.VMEM((CH, width), jnp.float32),
            pltpu.SemaphoreType.DMA,
            pltpu.SemaphoreType.DMA,
            pltpu.SemaphoreType.DMA,
            pltpu.SemaphoreType.DMA,
            pltpu.VMEM_SHARED((NPAD, width), jnp.float32),
        ],
        compiler_params=pltpu.CompilerParams(use_tc_tiling_on_sc=False),
    )
    def k(h_hbm, idx_hbm, out_hbm, src_v, dst_v, st_a, st_b,
          gs_a, gs_b, ss_a, ss_b, agg_sh):
        c = lax.axis_index("c")
        s = lax.axis_index("s")
        wid = c * NS + s
        qpr = width // 16

        @pl.loop(0, CH * qpr)
        def _(t):
            st_a[t // qpr, pl.ds((t % qpr) * 16, 16)] = (
                jnp.zeros((16,), jnp.float32))

        base = s * RPW

        @pl.loop(0, RPW // CH)
        def _(t):
            pltpu.sync_copy(st_a, agg_sh.at[pl.ds(base + t * CH, CH)])

        pltpu.sync_copy(idx_hbm.at[0, wid], src_v)
        pltpu.sync_copy(idx_hbm.at[1, wid], dst_v)
        plsc.subcore_barrier()

        pltpu.async_copy(h_hbm.at[src_v.at[0]], st_a, gs_a)

        @pl.loop(0, NCHUNK // 2)
        def _(p):
            j = 2 * p
            pltpu.make_async_copy(h_hbm.at[src_v.at[j]], st_a, gs_a).wait()
            pltpu.async_copy(h_hbm.at[src_v.at[j + 1]], st_b, gs_b)
            pltpu.async_copy(st_a, agg_sh.at[dst_v.at[j]], ss_a, add=True)
            pltpu.make_async_copy(h_hbm.at[src_v.at[j + 1]], st_b, gs_b).wait()
            pltpu.make_async_copy(st_a, agg_sh.at[dst_v.at[j]], ss_a).wait()

            @pl.when(j + 2 < NCHUNK)
            def _():
                pltpu.async_copy(h_hbm.at[src_v.at[j + 2]], st_a, gs_a)

            pltpu.async_copy(st_b, agg_sh.at[dst_v.at[j + 1]], ss_b, add=True)
            pltpu.make_async_copy(st_b, agg_sh.at[dst_v.at[j + 1]], ss_b).wait()

        plsc.subcore_barrier()
        pltpu.sync_copy(agg_sh.at[pl.ds(base, RPW)],
                        out_hbm.at[c, pl.ds(base, RPW)])

    return k(h, idx_all)


_ROWS = 400
_GRID = N // _ROWS


def _tc_layer1(x, w1, degp):
    def body(x_ref, w_ref, d_ref, o_ref):
        d = d_ref[0, 0] + d_ref[1, 0]
        nrm = lax.rsqrt(jnp.maximum(d, 1.0))
        o_ref[...] = jnp.dot(x_ref[...], w_ref[...],
                             preferred_element_type=jnp.float32,
                             precision=lax.Precision.HIGHEST) * nrm

    return pl.pallas_call(
        body,
        grid=(_GRID,),
        in_specs=[
            pl.BlockSpec((_ROWS, D_IN), lambda i: (i, 0)),
            pl.BlockSpec((D_IN, D_HID), lambda i: (0, 0)),
            pl.BlockSpec((2, 2, _ROWS, 1), lambda i: (0, 0, i, 0)),
        ],
        out_specs=pl.BlockSpec((_ROWS, D_HID), lambda i: (i, 0)),
        out_shape=jax.ShapeDtypeStruct((N, D_HID), jnp.float32),
    )(x, w1, degp)


def _tc_layer2(p1, degp, b1, w2):
    def body(p_ref, d_ref, b_ref, w_ref, o_ref):
        dout = d_ref[0, 0] + d_ref[1, 0]
        din = d_ref[0, 1] + d_ref[1, 1]
        nin = lax.rsqrt(jnp.maximum(din, 1.0))
        nout = lax.rsqrt(jnp.maximum(dout, 1.0))
        h = p_ref[0] + p_ref[1]
        h = jnp.maximum(h * nin + b_ref[...], 0.0)
        o_ref[...] = jnp.dot(h, w_ref[...],
                             preferred_element_type=jnp.float32,
                             precision=lax.Precision.HIGHEST) * nout

    return pl.pallas_call(
        body,
        grid=(_GRID,),
        in_specs=[
            pl.BlockSpec((2, _ROWS, D_HID), lambda i: (0, i, 0)),
            pl.BlockSpec((2, 2, _ROWS, 1), lambda i: (0, 0, i, 0)),
            pl.BlockSpec((1, D_HID), lambda i: (0, 0)),
            pl.BlockSpec((D_HID, D_OUT), lambda i: (0, 0)),
        ],
        out_specs=pl.BlockSpec((_ROWS, D_OUT), lambda i: (i, 0)),
        out_shape=jax.ShapeDtypeStruct((N, D_OUT), jnp.float32),
    )(p1, degp, b1, w2)


def _tc_final(p2, degp, b2):
    def body(p_ref, d_ref, b_ref, o_ref):
        din = d_ref[0, 1] + d_ref[1, 1]
        nin = lax.rsqrt(jnp.maximum(din, 1.0))
        o_ref[...] = (p_ref[0] + p_ref[1]) * nin + b_ref[...]

    return pl.pallas_call(
        body,
        grid=(_GRID,),
        in_specs=[
            pl.BlockSpec((2, _ROWS, D_OUT), lambda i: (0, i, 0)),
            pl.BlockSpec((2, 2, _ROWS, 1), lambda i: (0, 0, i, 0)),
            pl.BlockSpec((1, D_OUT), lambda i: (0, 0)),
        ],
        out_specs=pl.BlockSpec((_ROWS, D_OUT), lambda i: (i, 0)),
        out_shape=jax.ShapeDtypeStruct((N, D_OUT), jnp.float32),
    )(p2, degp, b2)


def kernel(features, edge_index, W1, b1, W2, b2):
    e = edge_index.astype(jnp.int32)
    src = e[0].reshape(NW, EPW)
    dst = e[1].reshape(NW, EPW)
    npad_e = EPAD - EPW
    pad_lanes = jnp.arange(npad_e, dtype=jnp.int32)
    pad_real = jnp.broadcast_to(pad_lanes % 16, (NW, npad_e))
    pad_dummy = jnp.broadcast_to(N + pad_lanes % (NPAD - N), (NW, npad_e))
    srcp = jnp.concatenate([src, pad_real], axis=1).reshape(NW, NCHUNK, CH)
    srcd = jnp.concatenate([src, pad_dummy], axis=1).reshape(NW, NCHUNK, CH)
    dstp = jnp.concatenate([dst, pad_dummy], axis=1).reshape(NW, NCHUNK, CH)
    idx_all = jnp.stack([srcp, dstp])
    idx_deg = jnp.stack([srcd, dstp])

    degp = _sc_degrees(idx_deg).reshape(2, 2, NPAD, 1)
    h1 = _tc_layer1(features, W1, degp)
    p1 = _sc_aggregate(h1, idx_all, D_HID)
    h2 = _tc_layer2(p1, degp, b1.reshape(1, D_HID), W2)
    p2 = _sc_aggregate(h2, idx_all, D_OUT)
    return _tc_final(p2, degp, b2.reshape(1, D_OUT))


# speedup vs baseline: 1.1459x; 1.1459x over previous
import functools

import jax
import jax.numpy as jnp
from jax import lax
from jax.experimental import pallas as pl
from jax.experimental.pallas import tpu as pltpu
from jax.experimental.pallas import tpu_sc as plsc

N = 10000
E = 320000
D_IN = 128
D_HID = 128
D_OUT = 16

NC, NS = 2, 16
NW = NC * NS
EPW = E // NW
CH = 128
NCHUNK = 80
EPAD = NCHUNK * CH
NPAD = 10240
RPW = NPAD // NS

_MESH = plsc.VectorSubcoreMesh(core_axis_name="c", subcore_axis_name="s")


def _sc_degrees(idx_all):
    @functools.partial(
        pl.kernel,
        out_type=jax.ShapeDtypeStruct((2, 2, NPAD), jnp.float32),
        mesh=_MESH,
        scratch_types=[
            pltpu.VMEM((NCHUNK, CH), jnp.int32),
            pltpu.VMEM((NCHUNK, CH), jnp.int32),
            pltpu.VMEM((CH,), jnp.float32),
            pltpu.VMEM((RPW,), jnp.float32),
            pltpu.VMEM_SHARED((NPAD,), jnp.float32),
            pltpu.VMEM_SHARED((NPAD,), jnp.float32),
        ],
    )
    def k(idx_hbm, out_hbm, src_v, dst_v, ones_v, zer_v, dsrc_sh, ddst_sh):
        c = lax.axis_index("c")
        s = lax.axis_index("s")
        wid = c * NS + s

        @pl.loop(0, CH // 16)
        def _(i):
            ones_v[pl.ds(i * 16, 16)] = jnp.ones((16,), jnp.float32)

        @pl.loop(0, RPW // 16)
        def _(i):
            zer_v[pl.ds(i * 16, 16)] = jnp.zeros((16,), jnp.float32)

        base = s * RPW
        pltpu.sync_copy(zer_v, dsrc_sh.at[pl.ds(base, RPW)])
        pltpu.sync_copy(zer_v, ddst_sh.at[pl.ds(base, RPW)])
        pltpu.sync_copy(idx_hbm.at[0, wid], src_v)
        pltpu.sync_copy(idx_hbm.at[1, wid], dst_v)
        plsc.subcore_barrier()

        @pl.loop(0, NCHUNK)
        def _(j):
            pltpu.sync_copy(ones_v, dsrc_sh.at[src_v.at[j]], add=True)
            pltpu.sync_copy(ones_v, ddst_sh.at[dst_v.at[j]], add=True)

        plsc.subcore_barrier()
        pltpu.sync_copy(dsrc_sh.at[pl.ds(base, RPW)],
                        out_hbm.at[c, 0, pl.ds(base, RPW)])
        pltpu.sync_copy(ddst_sh.at[pl.ds(base, RPW)],
                        out_hbm.at[c, 1, pl.ds(base, RPW)])

    return k(idx_all)


def _sc_aggregate(h, idx_all, width, pipelined):
    n_stage = 2 if pipelined else 1

    @functools.partial(
        pl.kernel,
        out_type=jax.ShapeDtypeStruct((2, NPAD, width), jnp.float32),
        mesh=_MESH,
        scratch_types=[
            pltpu.VMEM((NCHUNK, CH), jnp.int32),
            pltpu.VMEM((NCHUNK, CH), jnp.int32),
            pltpu.VMEM((n_stage, CH, width), jnp.float32),
            pltpu.SemaphoreType.DMA,
            pltpu.SemaphoreType.DMA,
            pltpu.SemaphoreType.DMA,
            pltpu.SemaphoreType.DMA,
            pltpu.VMEM_SHARED((NPAD, width), jnp.float32),
        ],
        compiler_params=pltpu.CompilerParams(use_tc_tiling_on_sc=False),
    )
    def k(h_hbm, idx_hbm, out_hbm, src_v, dst_v, st_v,
          gs_a, gs_b, ss_a, ss_b, agg_sh):
        c = lax.axis_index("c")
        s = lax.axis_index("s")
        wid = c * NS + s
        qpr = width // 16

        @pl.loop(0, CH * qpr)
        def _(t):
            st_v[0, t // qpr, pl.ds((t % qpr) * 16, 16)] = (
                jnp.zeros((16,), jnp.float32))

        base = s * RPW

        @pl.loop(0, RPW // CH)
        def _(t):
            pltpu.sync_copy(st_v.at[0], agg_sh.at[pl.ds(base + t * CH, CH)])

        pltpu.sync_copy(idx_hbm.at[0, wid], src_v)
        pltpu.sync_copy(idx_hbm.at[1, wid], dst_v)
        plsc.subcore_barrier()

        if not pipelined:
            @pl.loop(0, NCHUNK)
            def _(j):
                pltpu.sync_copy(h_hbm.at[src_v.at[j]], st_v.at[0])
                pltpu.sync_copy(st_v.at[0], agg_sh.at[dst_v.at[j]], add=True)
        else:
            st_a = st_v.at[0]
            st_b = st_v.at[1]
            pltpu.async_copy(h_hbm.at[src_v.at[0]], st_a, gs_a)

            @pl.loop(0, NCHUNK // 2)
            def _(p):
                j = 2 * p
                pltpu.make_async_copy(h_hbm.at[src_v.at[j]], st_a, gs_a).wait()
                pltpu.async_copy(h_hbm.at[src_v.at[j + 1]], st_b, gs_b)
                pltpu.async_copy(st_a, agg_sh.at[dst_v.at[j]], ss_a, add=True)
                pltpu.make_async_copy(
                    h_hbm.at[src_v.at[j + 1]], st_b, gs_b).wait()
                pltpu.make_async_copy(
                    st_a, agg_sh.at[dst_v.at[j]], ss_a).wait()

                @pl.when(j + 2 < NCHUNK)
                def _():
                    pltpu.async_copy(h_hbm.at[src_v.at[j + 2]], st_a, gs_a)

                pltpu.async_copy(
                    st_b, agg_sh.at[dst_v.at[j + 1]], ss_b, add=True)
                pltpu.make_async_copy(
                    st_b, agg_sh.at[dst_v.at[j + 1]], ss_b).wait()

        plsc.subcore_barrier()
        pltpu.sync_copy(agg_sh.at[pl.ds(base, RPW)],
                        out_hbm.at[c, pl.ds(base, RPW)])

    return k(h, idx_all)


_ROWS = 400
_GRID = N // _ROWS


def _tc_layer1(x, w1, degp):
    def body(x_ref, w_ref, d_ref, o_ref):
        d = d_ref[0, 0] + d_ref[1, 0]
        nrm = lax.rsqrt(jnp.maximum(d, 1.0))
        o_ref[...] = jnp.dot(x_ref[...], w_ref[...],
                             preferred_element_type=jnp.float32,
                             precision=lax.Precision.HIGHEST) * nrm

    return pl.pallas_call(
        body,
        grid=(_GRID,),
        in_specs=[
            pl.BlockSpec((_ROWS, D_IN), lambda i: (i, 0)),
            pl.BlockSpec((D_IN, D_HID), lambda i: (0, 0)),
            pl.BlockSpec((2, 2, _ROWS, 1), lambda i: (0, 0, i, 0)),
        ],
        out_specs=pl.BlockSpec((_ROWS, D_HID), lambda i: (i, 0)),
        out_shape=jax.ShapeDtypeStruct((N, D_HID), jnp.float32),
    )(x, w1, degp)


def _tc_layer2(p1, degp, b1, w2):
    def body(p_ref, d_ref, b_ref, w_ref, o_ref):
        dout = d_ref[0, 0] + d_ref[1, 0]
        din = d_ref[0, 1] + d_ref[1, 1]
        nin = lax.rsqrt(jnp.maximum(din, 1.0))
        nout = lax.rsqrt(jnp.maximum(dout, 1.0))
        h = p_ref[0] + p_ref[1]
        h = jnp.maximum(h * nin + b_ref[...], 0.0)
        o_ref[...] = jnp.dot(h, w_ref[...],
                             preferred_element_type=jnp.float32,
                             precision=lax.Precision.HIGHEST) * nout

    return pl.pallas_call(
        body,
        grid=(_GRID,),
        in_specs=[
            pl.BlockSpec((2, _ROWS, D_HID), lambda i: (0, i, 0)),
            pl.BlockSpec((2, 2, _ROWS, 1), lambda i: (0, 0, i, 0)),
            pl.BlockSpec((1, D_HID), lambda i: (0, 0)),
            pl.BlockSpec((D_HID, D_OUT), lambda i: (0, 0)),
        ],
        out_specs=pl.BlockSpec((_ROWS, D_OUT), lambda i: (i, 0)),
        out_shape=jax.ShapeDtypeStruct((N, D_OUT), jnp.float32),
    )(p1, degp, b1, w2)


def _tc_final(p2, degp, b2):
    def body(p_ref, d_ref, b_ref, o_ref):
        din = d_ref[0, 1] + d_ref[1, 1]
        nin = lax.rsqrt(jnp.maximum(din, 1.0))
        o_ref[...] = (p_ref[0] + p_ref[1]) * nin + b_ref[...]

    return pl.pallas_call(
        body,
        grid=(_GRID,),
        in_specs=[
            pl.BlockSpec((2, _ROWS, D_OUT), lambda i: (0, i, 0)),
            pl.BlockSpec((2, 2, _ROWS, 1), lambda i: (0, 0, i, 0)),
            pl.BlockSpec((1, D_OUT), lambda i: (0, 0)),
        ],
        out_specs=pl.BlockSpec((_ROWS, D_OUT), lambda i: (i, 0)),
        out_shape=jax.ShapeDtypeStruct((N, D_OUT), jnp.float32),
    )(p2, degp, b2)


def kernel(features, edge_index, W1, b1, W2, b2):
    e = edge_index.astype(jnp.int32)
    src = e[0].reshape(NW, EPW)
    dst = e[1].reshape(NW, EPW)
    npad_e = EPAD - EPW
    pad_lanes = jnp.arange(npad_e, dtype=jnp.int32)
    pad_real = jnp.broadcast_to(pad_lanes % 16, (NW, npad_e))
    pad_dummy = jnp.broadcast_to(N + pad_lanes % (NPAD - N), (NW, npad_e))
    srcp = jnp.concatenate([src, pad_real], axis=1).reshape(NW, NCHUNK, CH)
    srcd = jnp.concatenate([src, pad_dummy], axis=1).reshape(NW, NCHUNK, CH)
    dstp = jnp.concatenate([dst, pad_dummy], axis=1).reshape(NW, NCHUNK, CH)
    idx_all = jnp.stack([srcp, dstp])
    idx_deg = jnp.stack([srcd, dstp])

    degp = _sc_degrees(idx_deg).reshape(2, 2, NPAD, 1)
    h1 = _tc_layer1(features, W1, degp)
    p1 = _sc_aggregate(h1, idx_all, D_HID, pipelined=False)
    h2 = _tc_layer2(p1, degp, b1.reshape(1, D_HID), W2)
    p2 = _sc_aggregate(h2, idx_all, D_OUT, pipelined=True)
    return _tc_final(p2, degp, b2.reshape(1, D_OUT))


# speedup vs baseline: 1.2932x; 1.1285x over previous
import functools

import jax
import jax.numpy as jnp
from jax import lax
from jax.experimental import pallas as pl
from jax.experimental.pallas import tpu as pltpu
from jax.experimental.pallas import tpu_sc as plsc

N = 10000
E = 320000
D_IN = 128
D_HID = 128
D_OUT = 16

NC, NS = 2, 16
NW = NC * NS
EPW = E // NW
CH = 128
NCHUNK = 80
EPAD = NCHUNK * CH
NPAD = 10240
RPW = NPAD // NS

_MESH = plsc.VectorSubcoreMesh(core_axis_name="c", subcore_axis_name="s")


def _sc_degrees(idx_all):
    @functools.partial(
        pl.kernel,
        out_type=jax.ShapeDtypeStruct((2, 2, NPAD), jnp.float32),
        mesh=_MESH,
        scratch_types=[
            pltpu.VMEM((NCHUNK, CH), jnp.int32),
            pltpu.VMEM((NCHUNK, CH), jnp.int32),
            pltpu.VMEM((CH,), jnp.float32),
            pltpu.VMEM((RPW,), jnp.float32),
            pltpu.VMEM_SHARED((NPAD,), jnp.float32),
            pltpu.VMEM_SHARED((NPAD,), jnp.float32),
        ],
    )
    def k(idx_hbm, out_hbm, src_v, dst_v, ones_v, zer_v, dsrc_sh, ddst_sh):
        c = lax.axis_index("c")
        s = lax.axis_index("s")
        wid = c * NS + s

        @pl.loop(0, CH // 16)
        def _(i):
            ones_v[pl.ds(i * 16, 16)] = jnp.ones((16,), jnp.float32)

        @pl.loop(0, RPW // 16)
        def _(i):
            zer_v[pl.ds(i * 16, 16)] = jnp.zeros((16,), jnp.float32)

        base = s * RPW
        pltpu.sync_copy(zer_v, dsrc_sh.at[pl.ds(base, RPW)])
        pltpu.sync_copy(zer_v, ddst_sh.at[pl.ds(base, RPW)])
        pltpu.sync_copy(idx_hbm.at[0, wid], src_v)
        pltpu.sync_copy(idx_hbm.at[1, wid], dst_v)
        plsc.subcore_barrier()

        @pl.loop(0, NCHUNK)
        def _(j):
            pltpu.sync_copy(ones_v, dsrc_sh.at[src_v.at[j]], add=True)
            pltpu.sync_copy(ones_v, ddst_sh.at[dst_v.at[j]], add=True)

        plsc.subcore_barrier()
        pltpu.sync_copy(dsrc_sh.at[pl.ds(base, RPW)],
                        out_hbm.at[c, 0, pl.ds(base, RPW)])
        pltpu.sync_copy(ddst_sh.at[pl.ds(base, RPW)],
                        out_hbm.at[c, 1, pl.ds(base, RPW)])

    return k(idx_all)


def _sc_aggregate(h, idx_all, width, stage_table):
    rpt = N // NS

    @functools.partial(
        pl.kernel,
        out_type=jax.ShapeDtypeStruct((2, NPAD, width), jnp.float32),
        mesh=_MESH,
        scratch_types=[
            pltpu.VMEM((NCHUNK, CH), jnp.int32),
            pltpu.VMEM((NCHUNK, CH), jnp.int32),
            pltpu.VMEM((CH, width), jnp.float32),
            pltpu.VMEM_SHARED((N if stage_table else 1, width), jnp.float32),
            pltpu.VMEM_SHARED((NPAD, width), jnp.float32),
        ],
        compiler_params=pltpu.CompilerParams(use_tc_tiling_on_sc=False),
    )
    def k(h_hbm, idx_hbm, out_hbm, src_v, dst_v, st_v, hst_sh, agg_sh):
        c = lax.axis_index("c")
        s = lax.axis_index("s")
        wid = c * NS + s
        qpr = width // 16

        @pl.loop(0, CH * qpr)
        def _(t):
            st_v[t // qpr, pl.ds((t % qpr) * 16, 16)] = (
                jnp.zeros((16,), jnp.float32))

        base = s * RPW

        @pl.loop(0, RPW // CH)
        def _(t):
            pltpu.sync_copy(st_v, agg_sh.at[pl.ds(base + t * CH, CH)])

        if stage_table:
            pltpu.sync_copy(h_hbm.at[pl.ds(s * rpt, rpt)],
                            hst_sh.at[pl.ds(s * rpt, rpt)])
        pltpu.sync_copy(idx_hbm.at[0, wid], src_v)
        pltpu.sync_copy(idx_hbm.at[1, wid], dst_v)
        plsc.subcore_barrier()

        if stage_table:
            @pl.loop(0, NCHUNK)
            def _(j):
                pltpu.sync_copy(hst_sh.at[src_v.at[j]], st_v)
                pltpu.sync_copy(st_v, agg_sh.at[dst_v.at[j]], add=True)
        else:
            @pl.loop(0, NCHUNK)
            def _(j):
                pltpu.sync_copy(h_hbm.at[src_v.at[j]], st_v)
                pltpu.sync_copy(st_v, agg_sh.at[dst_v.at[j]], add=True)

        plsc.subcore_barrier()
        pltpu.sync_copy(agg_sh.at[pl.ds(base, RPW)],
                        out_hbm.at[c, pl.ds(base, RPW)])

    return k(h, idx_all)


_ROWS = 400
_GRID = N // _ROWS


def _tc_layer1(x, w1, degp):
    def body(x_ref, w_ref, d_ref, o_ref):
        d = d_ref[0, 0] + d_ref[1, 0]
        nrm = lax.rsqrt(jnp.maximum(d, 1.0))
        o_ref[...] = jnp.dot(x_ref[...], w_ref[...],
                             preferred_element_type=jnp.float32,
                             precision=lax.Precision.HIGHEST) * nrm

    return pl.pallas_call(
        body,
        grid=(_GRID,),
        in_specs=[
            pl.BlockSpec((_ROWS, D_IN), lambda i: (i, 0)),
            pl.BlockSpec((D_IN, D_HID), lambda i: (0, 0)),
            pl.BlockSpec((2, 2, _ROWS, 1), lambda i: (0, 0, i, 0)),
        ],
        out_specs=pl.BlockSpec((_ROWS, D_HID), lambda i: (i, 0)),
        out_shape=jax.ShapeDtypeStruct((N, D_HID), jnp.float32),
    )(x, w1, degp)


def _tc_layer2(p1, degp, b1, w2):
    def body(p_ref, d_ref, b_ref, w_ref, o_ref):
        dout = d_ref[0, 0] + d_ref[1, 0]
        din = d_ref[0, 1] + d_ref[1, 1]
        nin = lax.rsqrt(jnp.maximum(din, 1.0))
        nout = lax.rsqrt(jnp.maximum(dout, 1.0))
        h = p_ref[0] + p_ref[1]
        h = jnp.maximum(h * nin + b_ref[...], 0.0)
        o_ref[...] = jnp.dot(h, w_ref[...],
                             preferred_element_type=jnp.float32,
                             precision=lax.Precision.HIGHEST) * nout

    return pl.pallas_call(
        body,
        grid=(_GRID,),
        in_specs=[
            pl.BlockSpec((2, _ROWS, D_HID), lambda i: (0, i, 0)),
            pl.BlockSpec((2, 2, _ROWS, 1), lambda i: (0, 0, i, 0)),
            pl.BlockSpec((1, D_HID), lambda i: (0, 0)),
            pl.BlockSpec((D_HID, D_OUT), lambda i: (0, 0)),
        ],
        out_specs=pl.BlockSpec((_ROWS, D_OUT), lambda i: (i, 0)),
        out_shape=jax.ShapeDtypeStruct((N, D_OUT), jnp.float32),
    )(p1, degp, b1, w2)


def _tc_final(p2, degp, b2):
    def body(p_ref, d_ref, b_ref, o_ref):
        din = d_ref[0, 1] + d_ref[1, 1]
        nin = lax.rsqrt(jnp.maximum(din, 1.0))
        o_ref[...] = (p_ref[0] + p_ref[1]) * nin + b_ref[...]

    return pl.pallas_call(
        body,
        grid=(_GRID,),
        in_specs=[
            pl.BlockSpec((2, _ROWS, D_OUT), lambda i: (0, i, 0)),
            pl.BlockSpec((2, 2, _ROWS, 1), lambda i: (0, 0, i, 0)),
            pl.BlockSpec((1, D_OUT), lambda i: (0, 0)),
        ],
        out_specs=pl.BlockSpec((_ROWS, D_OUT), lambda i: (i, 0)),
        out_shape=jax.ShapeDtypeStruct((N, D_OUT), jnp.float32),
    )(p2, degp, b2)


def kernel(features, edge_index, W1, b1, W2, b2):
    e = edge_index.astype(jnp.int32)
    src = e[0].reshape(NW, EPW)
    dst = e[1].reshape(NW, EPW)
    npad_e = EPAD - EPW
    pad_lanes = jnp.arange(npad_e, dtype=jnp.int32)
    pad_real = jnp.broadcast_to(pad_lanes % 16, (NW, npad_e))
    pad_dummy = jnp.broadcast_to(N + pad_lanes % (NPAD - N), (NW, npad_e))
    srcp = jnp.concatenate([src, pad_real], axis=1).reshape(NW, NCHUNK, CH)
    srcd = jnp.concatenate([src, pad_dummy], axis=1).reshape(NW, NCHUNK, CH)
    dstp = jnp.concatenate([dst, pad_dummy], axis=1).reshape(NW, NCHUNK, CH)
    idx_all = jnp.stack([srcp, dstp])
    idx_deg = jnp.stack([srcd, dstp])

    degp = _sc_degrees(idx_deg).reshape(2, 2, NPAD, 1)
    h1 = _tc_layer1(features, W1, degp)
    p1 = _sc_aggregate(h1, idx_all, D_HID, stage_table=False)
    h2 = _tc_layer2(p1, degp, b1.reshape(1, D_HID), W2)
    p2 = _sc_aggregate(h2, idx_all, D_OUT, stage_table=True)
    return _tc_final(p2, degp, b2.reshape(1, D_OUT))
